# baseline (speedup 1.0000x reference)
import jax
import jax.numpy as jnp
from jax import lax
from jax.experimental import pallas as pl
from jax.experimental.pallas import tpu as pltpu

CAPACITY = 100000
IN_FEATURES = 512
OUT_FEATURES = 256
BLOCK = 8000
NBLK = (CAPACITY + BLOCK - 1) // BLOCK


def _body(q_ref, k_ref, v_ref, o_ref, m_ref, s_ref, acc_ref):
    i = pl.program_id(0)
    nblk = pl.num_programs(0)

    q = q_ref[...]
    k = k_ref[...]
    v = v_ref[...]

    d = -jnp.sum(jnp.abs(k - q), axis=1)
    if CAPACITY % BLOCK != 0:
        row = i * BLOCK + lax.broadcasted_iota(jnp.int32, (BLOCK,), 0)
        d = jnp.where(row < CAPACITY, d, -jnp.inf)
    m_blk = jnp.max(d)

    @pl.when(i == 0)
    def _init():
        m_ref[0] = m_blk
        s_ref[0] = 0.0
        acc_ref[...] = jnp.zeros_like(acc_ref)

    m_prev = m_ref[0]
    m_new = jnp.maximum(m_prev, m_blk)
    alpha = jnp.exp(m_prev - m_new)
    w = jnp.exp(d - m_new)
    s_ref[0] = s_ref[0] * alpha + jnp.sum(w)
    wv = lax.dot_general(
        w[None, :], v, (((1,), (0,)), ((), ())),
        preferred_element_type=jnp.float32)
    acc_ref[...] = acc_ref[...] * alpha + wv
    m_ref[0] = m_new

    @pl.when(i == nblk - 1)
    def _fin():
        o_ref[...] = acc_ref[...] / s_ref[0]


@jax.jit
def kernel(query, keys, values):
    out = pl.pallas_call(
        _body,
        grid=(NBLK,),
        in_specs=[
            pl.BlockSpec((1, IN_FEATURES), lambda i: (0, 0)),
            pl.BlockSpec((BLOCK, IN_FEATURES), lambda i: (i, 0)),
            pl.BlockSpec((BLOCK, OUT_FEATURES), lambda i: (i, 0)),
        ],
        out_specs=pl.BlockSpec((1, OUT_FEATURES), lambda i: (0, 0)),
        out_shape=jax.ShapeDtypeStruct((1, OUT_FEATURES), jnp.float32),
        scratch_shapes=[
            pltpu.SMEM((1,), jnp.float32),
            pltpu.SMEM((1,), jnp.float32),
            pltpu.VMEM((1, OUT_FEATURES), jnp.float32),
        ],
    )(query[None, :], keys, values)
    return out[0]


# speedup vs baseline: 1.0256x; 1.0256x over previous
import jax
import jax.numpy as jnp
from jax import lax
from jax.experimental import pallas as pl
from jax.experimental.pallas import tpu as pltpu

CAPACITY = 100000
IN_FEATURES = 512
OUT_FEATURES = 256
BLOCK = 6400
NBLK = (CAPACITY + BLOCK - 1) // BLOCK


def _body(q_ref, k_ref, v_ref, o_ref, m_ref, s_ref, acc_ref):
    i = pl.program_id(0)
    nblk = pl.num_programs(0)

    q = q_ref[...]
    k = k_ref[...]
    v = v_ref[...]

    d = -jnp.sum(jnp.abs(k - q), axis=1)
    if CAPACITY % BLOCK != 0:
        row = i * BLOCK + lax.broadcasted_iota(jnp.int32, (BLOCK,), 0)
        d = jnp.where(row < CAPACITY, d, -jnp.inf)
    m_blk = jnp.max(d)

    @pl.when(i == 0)
    def _init():
        m_ref[0] = m_blk
        s_ref[0] = 0.0
        acc_ref[...] = jnp.zeros_like(acc_ref)

    m_prev = m_ref[0]
    m_new = jnp.maximum(m_prev, m_blk)
    alpha = jnp.exp(m_prev - m_new)
    w = jnp.exp(d - m_new)
    s_ref[0] = s_ref[0] * alpha + jnp.sum(w)
    wv = lax.dot_general(
        w[None, :], v, (((1,), (0,)), ((), ())),
        preferred_element_type=jnp.float32)
    acc_ref[...] = acc_ref[...] * alpha + wv
    m_ref[0] = m_new

    @pl.when(i == nblk - 1)
    def _fin():
        o_ref[...] = acc_ref[...] / s_ref[0]


@jax.jit
def kernel(query, keys, values):
    out = pl.pallas_call(
        _body,
        grid=(NBLK,),
        in_specs=[
            pl.BlockSpec((1, IN_FEATURES), lambda i: (0, 0)),
            pl.BlockSpec((BLOCK, IN_FEATURES), lambda i: (i, 0)),
            pl.BlockSpec((BLOCK, OUT_FEATURES), lambda i: (i, 0)),
        ],
        out_specs=pl.BlockSpec((1, OUT_FEATURES), lambda i: (0, 0)),
        out_shape=jax.ShapeDtypeStruct((1, OUT_FEATURES), jnp.float32),
        scratch_shapes=[
            pltpu.SMEM((1,), jnp.float32),
            pltpu.SMEM((1,), jnp.float32),
            pltpu.VMEM((1, OUT_FEATURES), jnp.float32),
        ],
    )(query[None, :], keys, values)
    return out[0]
